# baseline (speedup 1.0000x reference)
import functools

import jax
import jax.numpy as jnp
import numpy as np
from jax import lax
from jax.experimental import pallas as pl
from jax.experimental.pallas import tpu as pltpu
from jax.experimental.pallas import tpu_sc as plsc

N = 10000
C = 256
AGG = 256
OUT = 256
S = 25

NUM_CORES = 2
NUM_SUBCORES = 16
NW = NUM_CORES * NUM_SUBCORES
ROWS_PER_W = (N + NW - 1) // NW
LANES = 16
NVREG_ROW = N // LANES
GS = 32
SENT = N

PACK_CC = 2048
PACK_W = 128
PACK_TAIL = N - 4 * PACK_CC
PROW = 5 * PACK_W
NWORDVREG = PROW // LANES


def _mk_pk(ncols):
    pk = np.zeros((ncols, PACK_W), np.float32)
    j = np.arange(ncols)
    pk[j, j // 16] = (2.0 ** (j % 16)).astype(np.float32)
    return pk


PK1_MAT = _mk_pk(PACK_CC)
PK2_MAT = _mk_pk(PACK_TAIL)


def _z_body(x_ref, w_ref, b_ref, z_ref):
    z_ref[...] = (
        jnp.dot(x_ref[...], w_ref[...], preferred_element_type=jnp.float32)
        + b_ref[...]
    )


def _z_matmul(x, w, b):
    blk = 1000
    return pl.pallas_call(
        _z_body,
        grid=(N // blk,),
        in_specs=[
            pl.BlockSpec((blk, C), lambda i: (i, 0)),
            pl.BlockSpec((C, AGG), lambda i: (0, 0)),
            pl.BlockSpec((1, AGG), lambda i: (0, 0)),
        ],
        out_specs=pl.BlockSpec((blk, AGG), lambda i: (i, 0)),
        out_shape=jax.ShapeDtypeStruct((N, AGG), jnp.float32),
    )(x, w, b)


def _pack_body(a_ref, pk1_ref, pk2_ref, p_ref):
    m = (a_ref[...] != 0.0).astype(jnp.bfloat16)
    for k in range(4):
        p_ref[:, k * PACK_W : (k + 1) * PACK_W] = jnp.dot(
            m[:, k * PACK_CC : (k + 1) * PACK_CC],
            pk1_ref[...],
            preferred_element_type=jnp.float32,
        ).astype(jnp.int32)
    p_ref[:, 4 * PACK_W : 5 * PACK_W] = jnp.dot(
        m[:, 4 * PACK_CC :],
        pk2_ref[...],
        preferred_element_type=jnp.float32,
    ).astype(jnp.int32)


def _pack(a, pk1, pk2):
    blk = 400
    return pl.pallas_call(
        _pack_body,
        grid=(N // blk,),
        in_specs=[
            pl.BlockSpec((blk, N), lambda i: (i, 0)),
            pl.BlockSpec((PACK_CC, PACK_W), lambda i: (0, 0)),
            pl.BlockSpec((PACK_TAIL, PACK_W), lambda i: (0, 0)),
        ],
        out_specs=pl.BlockSpec((blk, PROW), lambda i: (i, 0)),
        out_shape=jax.ShapeDtypeStruct((N, PROW), jnp.int32),
    )(a, pk1, pk2)


def _out_body(x_ref, a_ref, wt_ref, wb_ref, b_ref, u_ref, ss_ref):
    u = (
        jnp.dot(x_ref[...], wt_ref[...], preferred_element_type=jnp.float32)
        + jnp.dot(
            a_ref[...].astype(jnp.float32),
            wb_ref[...],
            preferred_element_type=jnp.float32,
        )
        + b_ref[...]
    )
    u = jnp.maximum(u, 0.0)
    u_ref[...] = u

    @pl.when(pl.program_id(0) == 0)
    def _():
        ss_ref[0, 0] = 0.0

    ss_ref[0, 0] += jnp.sum(u * u)


def _out_matmul(x, agg, wt, wb, b):
    blk = 1000
    return pl.pallas_call(
        _out_body,
        grid=(N // blk,),
        in_specs=[
            pl.BlockSpec((blk, C), lambda i: (i, 0)),
            pl.BlockSpec((blk, AGG), lambda i: (i, 0)),
            pl.BlockSpec((C, OUT), lambda i: (0, 0)),
            pl.BlockSpec((AGG, OUT), lambda i: (0, 0)),
            pl.BlockSpec((1, OUT), lambda i: (0, 0)),
        ],
        out_specs=[
            pl.BlockSpec((blk, OUT), lambda i: (i, 0)),
            pl.BlockSpec(memory_space=pltpu.SMEM),
        ],
        out_shape=[
            jax.ShapeDtypeStruct((N, OUT), jnp.float32),
            jax.ShapeDtypeStruct((1, 1), jnp.float32),
        ],
    )(x, agg, wt, wb, b)


def _scale_body(u_ref, d_ref, o_ref):
    o_ref[...] = u_ref[...] / d_ref[0, 0]


def _scale(u, denom):
    blk = 1000
    return pl.pallas_call(
        _scale_body,
        grid=(N // blk,),
        in_specs=[
            pl.BlockSpec((blk, OUT), lambda i: (i, 0)),
            pl.BlockSpec(memory_space=pltpu.SMEM),
        ],
        out_specs=pl.BlockSpec((blk, OUT), lambda i: (i, 0)),
        out_shape=jax.ShapeDtypeStruct((N, OUT), jnp.float32),
    )(u, denom)


NB = 4
GROUPS = (ROWS_PER_W + NB - 1) // NB


def _sc_agg_body(
    p_hbm, z_hbm, agg_hbm,
    prow_v, wq, idx_buf, rows_v, out_row, sem_g,
):
    cid = lax.axis_index("c")
    sid = lax.axis_index("s")
    wid = sid * NUM_CORES + cid
    base = wid * ROWS_PER_W

    sent = jnp.full((LANES,), SENT, jnp.int32)

    def scan_row(r, slot):
        pltpu.sync_copy(p_hbm.at[r], prow_v.at[pl.ds(0, PROW)])
        off = slot * GS
        idx_buf[pl.ds(off, LANES)] = sent
        idx_buf[pl.ds(off + 16, LANES)] = sent

        def wbody(vi, qn):
            w16 = prow_v[pl.ds(vi * LANES, LANES)]
            mw = w16 != 0
            nzw = plsc.all_reduce_population_count(mw)[0]
            active = jnp.logical_and(qn < S, nzw > 0)

            @pl.when(active)
            def _():
                widx = lax.iota(jnp.int32, LANES) + vi * LANES
                plsc.store_compressed(wq.at[pl.ds(qn, LANES)], widx, mask=mw)

            return qn + nzw * (qn < S).astype(jnp.int32)

        qn = lax.fori_loop(0, NWORDVREG, wbody, jnp.int32(0))

        def bbody(u, cnt):
            uvec = jnp.broadcast_to(u, (LANES,))
            wi_vec = plsc.load_gather(wq, [uvec])
            w_vec = plsc.load_gather(prow_v, [wi_vec])
            bits = lax.shift_right_logical(w_vec, lax.iota(jnp.int32, LANES))
            mb = lax.bitwise_and(bits, 1) != 0
            nb = plsc.all_reduce_population_count(mb)[0]

            @pl.when(cnt < S)
            def _():
                cols = lax.iota(jnp.int32, LANES) + wi_vec * LANES
                plsc.store_compressed(
                    idx_buf.at[pl.ds(off + cnt, LANES)], cols, mask=mb
                )

            return cnt + nb * (cnt < S).astype(jnp.int32)

        lax.fori_loop(0, jnp.minimum(qn, S), bbody, jnp.int32(0))

        lane = lax.iota(jnp.int32, LANES) + 16
        v1 = idx_buf[pl.ds(off + 16, LANES)]
        idx_buf[pl.ds(off + 16, LANES)] = jnp.where(lane < S, v1, sent)

    def reduce_row(r, slot):
        rbase = slot * GS
        zero = jnp.zeros((2 * LANES,), jnp.bfloat16)
        for v in range(AGG // (2 * LANES)):
            acc = plsc.bitcast(
                rows_v[rbase, pl.ds(v * LANES, LANES)], jnp.bfloat16
            )
            for si in range(1, GS):
                acc = jnp.maximum(
                    acc,
                    plsc.bitcast(
                        rows_v[rbase + si, pl.ds(v * LANES, LANES)],
                        jnp.bfloat16,
                    ),
                )
            out_row[pl.ds(v * LANES, LANES)] = plsc.bitcast(
                jnp.maximum(acc, zero), jnp.int32
            )
        pltpu.sync_copy(out_row, agg_hbm.at[r])

    def group_body(g, carry):
        r0 = base + NB * g

        for b in range(NB):
            ok = jnp.logical_and(NB * g + b < ROWS_PER_W, r0 + b < N)

            @pl.when(ok)
            def _(b=b):
                scan_row(r0 + b, b)

            @pl.when(jnp.logical_not(ok))
            def _(b=b):
                idx_buf[pl.ds(b * GS, LANES)] = sent
                idx_buf[pl.ds(b * GS + 16, LANES)] = sent

        pltpu.async_copy(z_hbm.at[idx_buf], rows_v, sem_g).wait()

        for b in range(NB):
            ok = jnp.logical_and(NB * g + b < ROWS_PER_W, r0 + b < N)

            @pl.when(ok)
            def _(b=b):
                reduce_row(r0 + b, b)

        return carry

    lax.fori_loop(0, GROUPS, group_body, jnp.int32(0))


_sc_agg = functools.partial(
    pl.kernel,
    out_type=jax.ShapeDtypeStruct((N, AGG // 2), jnp.int32),
    mesh=plsc.VectorSubcoreMesh(
        core_axis_name="c",
        subcore_axis_name="s",
        num_cores=NUM_CORES,
        num_subcores=NUM_SUBCORES,
    ),
    scratch_types=[
        pltpu.VMEM((PROW + LANES,), jnp.int32),
        pltpu.VMEM((64,), jnp.int32),
        pltpu.VMEM((NB * GS,), jnp.int32),
        pltpu.VMEM((NB * GS, AGG // 2), jnp.int32),
        pltpu.VMEM((AGG // 2,), jnp.int32),
        pltpu.SemaphoreType.DMA,
    ],
    compiler_params=pltpu.CompilerParams(needs_layout_passes=False),
)(_sc_agg_body)


@jax.jit
def kernel(A, X, W_agg, b_agg, W_cat, b_cat):
    a = A.reshape(N, N)
    x = X.reshape(N, C)
    p = _pack(
        a,
        jnp.asarray(PK1_MAT).astype(jnp.bfloat16),
        jnp.asarray(PK2_MAT).astype(jnp.bfloat16),
    )
    z = _z_matmul(x, W_agg, b_agg.reshape(1, AGG))
    ztab = lax.bitcast_convert_type(
        jnp.concatenate([z, b_agg.reshape(1, AGG)], axis=0)
        .astype(jnp.bfloat16)
        .reshape(N + 1, AGG // 2, 2),
        jnp.int32,
    )
    agg = lax.bitcast_convert_type(_sc_agg(p, ztab), jnp.bfloat16).reshape(
        N, AGG
    )
    u, ss = _out_matmul(
        x, agg, W_cat[:C], W_cat[C:], b_cat.reshape(1, OUT)
    )
    norm = jnp.sqrt(ss[0, 0])
    denom = jnp.where(norm == 0.0, 1.0, norm).reshape(1, 1)
    return _scale(u, denom).reshape(1, N, OUT)


# speedup vs baseline: 3.1389x; 3.1389x over previous
import functools

import jax
import jax.numpy as jnp
import numpy as np
from jax import lax
from jax.experimental import pallas as pl
from jax.experimental.pallas import tpu as pltpu
from jax.experimental.pallas import tpu_sc as plsc

N = 10000
C = 256
AGG = 256
OUT = 256
S = 25

NUM_CORES = 2
NUM_SUBCORES = 16
NW = NUM_CORES * NUM_SUBCORES
ROWS_PER_W = (N + NW - 1) // NW
LANES = 16
NVREG_ROW = N // LANES
GS = 32
SENT = N

PACK_CC = 2048
PACK_W = 128
PACK_TAIL = N - 4 * PACK_CC
PROW = 5 * PACK_W
NWORDVREG = PROW // LANES


def _mk_pk(ncols):
    pk = np.zeros((ncols, PACK_W), np.float32)
    j = np.arange(ncols)
    pk[j, j // 16] = (2.0 ** (j % 16)).astype(np.float32)
    return pk


PK1_MAT = _mk_pk(PACK_CC)
PK2_MAT = _mk_pk(PACK_TAIL)


def _z_body(x_ref, w_ref, b_ref, z_ref):
    z_ref[...] = (
        jnp.dot(x_ref[...], w_ref[...], preferred_element_type=jnp.float32)
        + b_ref[...]
    )


def _z_matmul(x, w, b):
    blk = 1000
    return pl.pallas_call(
        _z_body,
        grid=(N // blk,),
        in_specs=[
            pl.BlockSpec((blk, C), lambda i: (i, 0)),
            pl.BlockSpec((C, AGG), lambda i: (0, 0)),
            pl.BlockSpec((1, AGG), lambda i: (0, 0)),
        ],
        out_specs=pl.BlockSpec((blk, AGG), lambda i: (i, 0)),
        out_shape=jax.ShapeDtypeStruct((N, AGG), jnp.float32),
    )(x, w, b)


def _pack_body(a_ref, pk1_ref, pk2_ref, p_ref):
    m = (a_ref[...] != 0.0).astype(jnp.bfloat16)
    for k in range(4):
        p_ref[:, k * PACK_W : (k + 1) * PACK_W] = jnp.dot(
            m[:, k * PACK_CC : (k + 1) * PACK_CC],
            pk1_ref[...],
            preferred_element_type=jnp.float32,
        ).astype(jnp.int32)
    p_ref[:, 4 * PACK_W : 5 * PACK_W] = jnp.dot(
        m[:, 4 * PACK_CC :],
        pk2_ref[...],
        preferred_element_type=jnp.float32,
    ).astype(jnp.int32)


def _pack(a, pk1, pk2):
    blk = 400
    return pl.pallas_call(
        _pack_body,
        grid=(N // blk,),
        in_specs=[
            pl.BlockSpec((blk, N), lambda i: (i, 0)),
            pl.BlockSpec((PACK_CC, PACK_W), lambda i: (0, 0)),
            pl.BlockSpec((PACK_TAIL, PACK_W), lambda i: (0, 0)),
        ],
        out_specs=pl.BlockSpec((blk, PROW), lambda i: (i, 0)),
        out_shape=jax.ShapeDtypeStruct((N, PROW), jnp.int32),
    )(a, pk1, pk2)


def _out_body(x_ref, a_ref, wt_ref, wb_ref, b_ref, u_ref, ss_ref):
    u = (
        jnp.dot(x_ref[...], wt_ref[...], preferred_element_type=jnp.float32)
        + jnp.dot(
            a_ref[...].astype(jnp.float32),
            wb_ref[...],
            preferred_element_type=jnp.float32,
        )
        + b_ref[...]
    )
    u = jnp.maximum(u, 0.0)
    u_ref[...] = u

    @pl.when(pl.program_id(0) == 0)
    def _():
        ss_ref[0, 0] = 0.0

    ss_ref[0, 0] += jnp.sum(u * u)


def _out_matmul(x, agg, wt, wb, b):
    blk = 1000
    return pl.pallas_call(
        _out_body,
        grid=(N // blk,),
        in_specs=[
            pl.BlockSpec((blk, C), lambda i: (i, 0)),
            pl.BlockSpec((blk, AGG), lambda i: (i, 0)),
            pl.BlockSpec((C, OUT), lambda i: (0, 0)),
            pl.BlockSpec((AGG, OUT), lambda i: (0, 0)),
            pl.BlockSpec((1, OUT), lambda i: (0, 0)),
        ],
        out_specs=[
            pl.BlockSpec((blk, OUT), lambda i: (i, 0)),
            pl.BlockSpec(memory_space=pltpu.SMEM),
        ],
        out_shape=[
            jax.ShapeDtypeStruct((N, OUT), jnp.float32),
            jax.ShapeDtypeStruct((1, 1), jnp.float32),
        ],
    )(x, agg, wt, wb, b)


def _scale_body(u_ref, d_ref, o_ref):
    o_ref[...] = u_ref[...] / d_ref[0, 0]


def _scale(u, denom):
    blk = 1000
    return pl.pallas_call(
        _scale_body,
        grid=(N // blk,),
        in_specs=[
            pl.BlockSpec((blk, OUT), lambda i: (i, 0)),
            pl.BlockSpec(memory_space=pltpu.SMEM),
        ],
        out_specs=pl.BlockSpec((blk, OUT), lambda i: (i, 0)),
        out_shape=jax.ShapeDtypeStruct((N, OUT), jnp.float32),
    )(u, denom)


NB = 4
GROUPS = (ROWS_PER_W + NB - 1) // NB


def _sc_agg_body(
    p_hbm, z_hbm, agg_hbm,
    prow_v, wq, idx_buf, rows_v, out_row, zsh, sem_g,
):
    cid = lax.axis_index("c")
    sid = lax.axis_index("s")
    wid = sid * NUM_CORES + cid
    base = wid * ROWS_PER_W

    @pl.when(sid == 0)
    def _():
        pltpu.sync_copy(z_hbm, zsh)

    plsc.subcore_barrier()

    sent = jnp.full((LANES,), SENT, jnp.int32)

    def scan_row(r, slot):
        pltpu.sync_copy(p_hbm.at[r], prow_v.at[pl.ds(0, PROW)])
        off = slot * GS
        idx_buf[pl.ds(off, LANES)] = sent
        idx_buf[pl.ds(off + 16, LANES)] = sent

        def wbody(vi, qn):
            w16 = prow_v[pl.ds(vi * LANES, LANES)]
            mw = w16 != 0
            nzw = plsc.all_reduce_population_count(mw)[0]
            active = jnp.logical_and(qn < S, nzw > 0)

            @pl.when(active)
            def _():
                widx = lax.iota(jnp.int32, LANES) + vi * LANES
                plsc.store_compressed(wq.at[pl.ds(qn, LANES)], widx, mask=mw)

            return qn + nzw * (qn < S).astype(jnp.int32)

        qn = lax.fori_loop(0, NWORDVREG, wbody, jnp.int32(0))

        def bbody(u, cnt):
            uvec = jnp.broadcast_to(u, (LANES,))
            wi_vec = plsc.load_gather(wq, [uvec])
            w_vec = plsc.load_gather(prow_v, [wi_vec])
            bits = lax.shift_right_logical(w_vec, lax.iota(jnp.int32, LANES))
            mb = lax.bitwise_and(bits, 1) != 0
            nb = plsc.all_reduce_population_count(mb)[0]

            @pl.when(cnt < S)
            def _():
                cols = lax.iota(jnp.int32, LANES) + wi_vec * LANES
                plsc.store_compressed(
                    idx_buf.at[pl.ds(off + cnt, LANES)], cols, mask=mb
                )

            return cnt + nb * (cnt < S).astype(jnp.int32)

        lax.fori_loop(0, jnp.minimum(qn, S), bbody, jnp.int32(0))

        lane = lax.iota(jnp.int32, LANES) + 16
        v1 = idx_buf[pl.ds(off + 16, LANES)]
        idx_buf[pl.ds(off + 16, LANES)] = jnp.where(lane < S, v1, sent)

    def reduce_row(r, slot):
        rbase = slot * GS
        zero = jnp.zeros((2 * LANES,), jnp.bfloat16)
        for v in range(AGG // (2 * LANES)):
            acc = plsc.bitcast(
                rows_v[rbase, pl.ds(v * LANES, LANES)], jnp.bfloat16
            )
            for si in range(1, GS):
                acc = jnp.maximum(
                    acc,
                    plsc.bitcast(
                        rows_v[rbase + si, pl.ds(v * LANES, LANES)],
                        jnp.bfloat16,
                    ),
                )
            out_row[pl.ds(v * LANES, LANES)] = plsc.bitcast(
                jnp.maximum(acc, zero), jnp.int32
            )
        pltpu.sync_copy(out_row, agg_hbm.at[r])

    def group_body(g, carry):
        r0 = base + NB * g

        for b in range(NB):
            ok = jnp.logical_and(NB * g + b < ROWS_PER_W, r0 + b < N)

            @pl.when(ok)
            def _(b=b):
                scan_row(r0 + b, b)

            @pl.when(jnp.logical_not(ok))
            def _(b=b):
                idx_buf[pl.ds(b * GS, LANES)] = sent
                idx_buf[pl.ds(b * GS + 16, LANES)] = sent

        pltpu.async_copy(zsh.at[idx_buf], rows_v, sem_g).wait()

        for b in range(NB):
            ok = jnp.logical_and(NB * g + b < ROWS_PER_W, r0 + b < N)

            @pl.when(ok)
            def _(b=b):
                reduce_row(r0 + b, b)

        return carry

    lax.fori_loop(0, GROUPS, group_body, jnp.int32(0))


_sc_agg = functools.partial(
    pl.kernel,
    out_type=jax.ShapeDtypeStruct((N, AGG // 2), jnp.int32),
    mesh=plsc.VectorSubcoreMesh(
        core_axis_name="c",
        subcore_axis_name="s",
        num_cores=NUM_CORES,
        num_subcores=NUM_SUBCORES,
    ),
    scratch_types=[
        pltpu.VMEM((PROW + LANES,), jnp.int32),
        pltpu.VMEM((64,), jnp.int32),
        pltpu.VMEM((NB * GS,), jnp.int32),
        pltpu.VMEM((NB * GS, AGG // 2), jnp.int32),
        pltpu.VMEM((AGG // 2,), jnp.int32),
        pltpu.VMEM_SHARED((N + 1, AGG // 2), jnp.int32),
        pltpu.SemaphoreType.DMA,
    ],
    compiler_params=pltpu.CompilerParams(needs_layout_passes=False),
)(_sc_agg_body)


@jax.jit
def kernel(A, X, W_agg, b_agg, W_cat, b_cat):
    a = A.reshape(N, N)
    x = X.reshape(N, C)
    p = _pack(
        a,
        jnp.asarray(PK1_MAT).astype(jnp.bfloat16),
        jnp.asarray(PK2_MAT).astype(jnp.bfloat16),
    )
    z = _z_matmul(x, W_agg, b_agg.reshape(1, AGG))
    ztab = lax.bitcast_convert_type(
        jnp.concatenate([z, b_agg.reshape(1, AGG)], axis=0)
        .astype(jnp.bfloat16)
        .reshape(N + 1, AGG // 2, 2),
        jnp.int32,
    )
    agg = lax.bitcast_convert_type(_sc_agg(p, ztab), jnp.bfloat16).reshape(
        N, AGG
    )
    u, ss = _out_matmul(
        x, agg, W_cat[:C], W_cat[C:], b_cat.reshape(1, OUT)
    )
    norm = jnp.sqrt(ss[0, 0])
    denom = jnp.where(norm == 0.0, 1.0, norm).reshape(1, 1)
    return _scale(u, denom).reshape(1, N, OUT)
